# baseline (speedup 1.0000x reference)
import functools

import jax
import jax.numpy as jnp
from jax import lax
from jax.experimental import pallas as pl
from jax.experimental.pallas import tpu as pltpu
from jax.experimental.pallas import tpu_sc as plsc

IN_F = 2048
E = 16
CHUNK = 1024
H = CHUNK // 2
NBUF = 4
NW = 32


def _tc_body(x_hbm, w_ref, lg_hbm, xbuf, lbuf, sems, osems):
    two, half, F = x_hbm.shape
    T = two * half
    nchunk = T // CHUNK
    w = w_ref[...]

    def in_copies(i, slot):
        return (
            pltpu.make_async_copy(
                x_hbm.at[:, pl.ds(i * H, H), :], xbuf.at[slot],
                sems.at[slot],
            ),
        )

    def out_copies(i, slot):
        return (
            pltpu.make_async_copy(
                lbuf.at[slot, pl.ds(0, H)], lg_hbm.at[pl.ds(i * H, H), :],
                osems.at[slot, 0],
            ),
            pltpu.make_async_copy(
                lbuf.at[slot, pl.ds(H, H)], lg_hbm.at[pl.ds(half + i * H, H), :],
                osems.at[slot, 1],
            ),
        )

    for b in range(NBUF):
        for c in in_copies(b, b):
            c.start()

    def step(i, carry):
        slot = jax.lax.rem(i, NBUF)
        for c in in_copies(i, slot):
            c.wait()
        x = xbuf[slot].reshape(CHUNK, IN_F)
        logits = jnp.dot(x, w, preferred_element_type=jnp.float32)

        @pl.when(i + NBUF < nchunk)
        def _():
            for c in in_copies(i + NBUF, slot):
                c.start()

        @pl.when(i >= NBUF)
        def _():
            for c in out_copies(i - NBUF, slot):
                c.wait()

        lbuf[slot] = logits
        for c in out_copies(i, slot):
            c.start()
        return carry

    jax.lax.fori_loop(0, nchunk, step, 0)
    for b in range(NBUF):
        i = nchunk - NBUF + b
        for c in out_copies(i, jax.lax.rem(i, NBUF)):
            c.wait()


def _tc_logits(x2, w128):
    two, half, F = x2.shape
    T = two * half
    return pl.pallas_call(
        _tc_body,
        in_specs=[
            pl.BlockSpec(memory_space=pltpu.MemorySpace.HBM),
            pl.BlockSpec(memory_space=pltpu.VMEM),
        ],
        out_specs=pl.BlockSpec(memory_space=pltpu.MemorySpace.HBM),
        out_shape=jax.ShapeDtypeStruct((T, 128), jnp.float32),
        scratch_shapes=[
            pltpu.VMEM((NBUF, 2, H, IN_F), jnp.float32),
            pltpu.VMEM((NBUF, CHUNK, 128), jnp.float32),
            pltpu.SemaphoreType.DMA((NBUF,)),
            pltpu.SemaphoreType.DMA((NBUF, 2)),
        ],
    )(x2, w128)


def _sc_router(lg):
    T = lg.shape[0]
    tw = T // NW
    mesh = plsc.VectorSubcoreMesh(core_axis_name="c", subcore_axis_name="s")

    @functools.partial(
        pl.kernel,
        mesh=mesh,
        out_type=jax.ShapeDtypeStruct((T, 128), jnp.float32),
        scratch_types=[
            pltpu.VMEM((tw, 128), jnp.float32),
        ],
    )
    def k(lg_hbm, out_hbm, slab):
        wid = lax.axis_index("s") * 2 + lax.axis_index("c")
        base = wid * tw
        pltpu.sync_copy(lg_hbm.at[pl.ds(base, tw), :], slab)
        lanes = lax.broadcasted_iota(jnp.int32, (16,), 0)
        dn = lax.GatherDimensionNumbers(
            offset_dims=(), collapsed_slice_dims=(0,), start_index_map=(0,))
        perms = [(lanes ^ s)[:, None] for s in (8, 4, 2, 1)]
        ninf = jnp.full((16,), -jnp.inf, jnp.float32)

        def bmax(v):
            for p in perms:
                v = jnp.maximum(v, lax.gather(
                    v, p, dn, (1,),
                    mode=lax.GatherScatterMode.PROMISE_IN_BOUNDS))
            return v

        def token(t, carry):
            logits = slab[t, pl.ds(0, 16)]
            li = lax.bitcast_convert_type(logits, jnp.int32)
            key_i = jnp.where(
                li >= 0, (li | 15) - lanes, (li & ~jnp.int32(15)) | lanes
            )
            keys = lax.bitcast_convert_type(key_i, jnp.float32)
            m1v = bmax(keys)
            m2v = bmax(jnp.where(keys == m1v, ninf, keys))
            m1i = lax.bitcast_convert_type(m1v, jnp.int32)
            m2i = lax.bitcast_convert_type(m2v, jnp.int32)
            i1 = jnp.where(m1i >= 0, 15 - (m1i & 15), m1i & 15)
            i2 = jnp.where(m2i >= 0, 15 - (m2i & 15), m2i & 15)
            e1 = jnp.exp(m2v - m1v)
            s = 1.0 + e1
            g0 = 1.0 / s
            g1 = e1 / s
            i1f = lax.bitcast_convert_type(i1, jnp.float32)
            i2f = lax.bitcast_convert_type(i2, jnp.float32)
            res = jnp.where(
                lanes == 0, g0,
                jnp.where(lanes == 1, g1,
                          jnp.where(lanes == 2, i1f,
                                    jnp.where(lanes == 3, i2f, g0))),
            )
            slab[t, pl.ds(0, 16)] = res
            return carry

        lax.fori_loop(0, tw, token, 0)
        pltpu.sync_copy(slab, out_hbm.at[pl.ds(base, tw), :])

    return k(lg)


@functools.partial(jax.jit, static_argnames=())
def kernel(x, weight):
    B, S, F = x.shape
    T = B * S
    x2 = x.reshape(2, T // 2, F)
    w128 = jnp.pad(weight, ((0, 0), (0, 128 - E)))
    lg = _tc_logits(x2, w128)
    packed = _sc_router(lg)
    gates = packed[:, 0:2]
    idx = lax.bitcast_convert_type(packed[:, 2:4], jnp.int32)
    return gates.reshape(B, S, 2), idx.reshape(B, S, 2)


# speedup vs baseline: 1.7285x; 1.7285x over previous
import functools

import jax
import jax.numpy as jnp
from jax.experimental import pallas as pl
from jax.experimental.pallas import tpu as pltpu

IN_F = 2048
E = 16
TBLK = 2048


def _top2(logits):
    lanes = jax.lax.broadcasted_iota(jnp.int32, logits.shape, 1)
    li = jax.lax.bitcast_convert_type(logits, jnp.int32)
    key_i = jnp.where(li >= 0, (li | 15) - lanes, (li & ~jnp.int32(15)) | lanes)
    keys = jax.lax.bitcast_convert_type(key_i, jnp.float32)
    m1 = jnp.max(keys, axis=-1, keepdims=True)
    masked = jnp.where(keys == m1, -jnp.inf, keys)
    m2 = jnp.max(masked, axis=-1, keepdims=True)

    def decode(m):
        mi = jax.lax.bitcast_convert_type(m, jnp.int32)
        payload = mi & 15
        return jnp.where(mi >= 0, 15 - payload, payload)

    e1 = jnp.exp(m2 - m1)
    s = 1.0 + e1
    return (jnp.concatenate([1.0 / s, e1 / s], axis=-1),
            jnp.concatenate([decode(m1), decode(m2)], axis=-1))


def _body(x_ref, w_ref, g_ref, i_ref):
    logits = jnp.dot(x_ref[...], w_ref[...], preferred_element_type=jnp.float32)
    g, ix = _top2(logits)
    g_ref[...] = g
    i_ref[...] = ix


@functools.partial(jax.jit, static_argnames=())
def kernel(x, weight):
    B, S, F = x.shape
    T = B * S
    x2 = x.reshape(T, F)
    grid = (T // TBLK,)
    gates, idx = pl.pallas_call(
        _body,
        grid=grid,
        in_specs=[
            pl.BlockSpec((TBLK, F), lambda i: (i, 0)),
            pl.BlockSpec((F, E), lambda i: (0, 0)),
        ],
        out_specs=[
            pl.BlockSpec((TBLK, 2), lambda i: (i, 0)),
            pl.BlockSpec((TBLK, 2), lambda i: (i, 0)),
        ],
        out_shape=[
            jax.ShapeDtypeStruct((T, 2), jnp.float32),
            jax.ShapeDtypeStruct((T, 2), jnp.int32),
        ],
        compiler_params=pltpu.CompilerParams(
            dimension_semantics=("arbitrary",),
        ),
    )(x2, weight)
    return gates.reshape(B, S, 2), idx.reshape(B, S, 2)
